# Initial kernel scaffold; baseline (speedup 1.0000x reference)
import jax
import jax.numpy as jnp
from jax.experimental import pallas as pl


def kernel(x, adj, W1, b1, W2, b2):
    raise NotImplementedError("write your pallas kernel here")


# speedup vs baseline: 1.0710x; 1.0710x over previous
import jax
import jax.numpy as jnp
from jax.experimental import pallas as pl

_BF = jnp.bfloat16
_F32 = jnp.float32
_MM_DIMS = (((1,), (0,)), ((), ()))


def _xw1_kernel(x_ref, w1_ref, s1_ref):
    s1_ref[...] = jax.lax.dot_general(
        x_ref[...].astype(_BF), w1_ref[...],
        _MM_DIMS, preferred_element_type=_F32).astype(_BF)


def _layer1_kernel(adj_ref, s1_ref, b1_ref, w2_ref, h_ref, s2_ref):
    acc = jax.lax.dot_general(
        adj_ref[...].astype(_BF), s1_ref[...],
        _MM_DIMS, preferred_element_type=_F32)
    h = jnp.maximum(acc + b1_ref[...], 0.0)
    h_ref[...] = h
    s2_ref[...] = jax.lax.dot_general(
        h.astype(_BF), w2_ref[...],
        _MM_DIMS, preferred_element_type=_F32).astype(_BF)


def _layer2_kernel(adj_ref, s2_ref, b2_ref, logits_ref, logp_ref):
    logits = jax.lax.dot_general(
        adj_ref[...].astype(_BF), s2_ref[...],
        _MM_DIMS, preferred_element_type=_F32) + b2_ref[...]
    m = jnp.max(logits, axis=1, keepdims=True)
    lse = m + jnp.log(jnp.sum(jnp.exp(logits - m), axis=1, keepdims=True))
    logits_ref[...] = logits
    logp_ref[...] = logits - lse


def kernel(x, adj, W1, b1, W2, b2):
    n, nfeat = x.shape
    nhid = W1.shape[1]
    nclass = W2.shape[1]

    bm1 = 1024
    s1 = pl.pallas_call(
        _xw1_kernel,
        grid=(n // bm1,),
        in_specs=[
            pl.BlockSpec((bm1, nfeat), lambda i: (i, 0)),
            pl.BlockSpec((nfeat, nhid), lambda i: (0, 0)),
        ],
        out_specs=pl.BlockSpec((bm1, nhid), lambda i: (i, 0)),
        out_shape=jax.ShapeDtypeStruct((n, nhid), _BF),
    )(x, W1.astype(_BF))

    bm = 256
    h, s2 = pl.pallas_call(
        _layer1_kernel,
        grid=(n // bm,),
        in_specs=[
            pl.BlockSpec((bm, n), lambda i: (i, 0)),
            pl.BlockSpec((n, nhid), lambda i: (0, 0)),
            pl.BlockSpec((1, nhid), lambda i: (0, 0)),
            pl.BlockSpec((nhid, nclass), lambda i: (0, 0)),
        ],
        out_specs=[
            pl.BlockSpec((bm, nhid), lambda i: (i, 0)),
            pl.BlockSpec((bm, nclass), lambda i: (i, 0)),
        ],
        out_shape=[
            jax.ShapeDtypeStruct((n, nhid), _F32),
            jax.ShapeDtypeStruct((n, nclass), _BF),
        ],
    )(adj, s1, b1.reshape(1, nhid), W2.astype(_BF))

    logits, logp = pl.pallas_call(
        _layer2_kernel,
        grid=(n // bm,),
        in_specs=[
            pl.BlockSpec((bm, n), lambda i: (i, 0)),
            pl.BlockSpec((n, nclass), lambda i: (0, 0)),
            pl.BlockSpec((1, nclass), lambda i: (0, 0)),
        ],
        out_specs=[
            pl.BlockSpec((bm, nclass), lambda i: (i, 0)),
            pl.BlockSpec((bm, nclass), lambda i: (i, 0)),
        ],
        out_shape=[
            jax.ShapeDtypeStruct((n, nclass), _F32),
            jax.ShapeDtypeStruct((n, nclass), _F32),
        ],
    )(adj, s2, b2.reshape(1, nclass))

    return (logp, logits, h)


# speedup vs baseline: 1.0741x; 1.0029x over previous
import jax
import jax.numpy as jnp
from jax.experimental import pallas as pl

_BF = jnp.bfloat16
_F32 = jnp.float32
_MM_DIMS = (((1,), (0,)), ((), ()))


def _xw1_kernel(x_ref, w1_ref, s1_ref):
    s1_ref[...] = jax.lax.dot_general(
        x_ref[...].astype(_BF), w1_ref[...],
        _MM_DIMS, preferred_element_type=_F32).astype(_BF)


def _layer1_kernel(adj_ref, s1_ref, b1_ref, w2_ref, h_ref, s2_ref):
    acc = jax.lax.dot_general(
        adj_ref[...].astype(_BF), s1_ref[...],
        _MM_DIMS, preferred_element_type=_F32)
    h = jnp.maximum(acc + b1_ref[...], 0.0)
    h_ref[...] = h
    s2_ref[...] = jax.lax.dot_general(
        h.astype(_BF), w2_ref[...],
        _MM_DIMS, preferred_element_type=_F32).astype(_BF)


def _layer2_kernel(adj_ref, s2_ref, b2_ref, logits_ref, logp_ref):
    logits = jax.lax.dot_general(
        adj_ref[...].astype(_BF), s2_ref[...],
        _MM_DIMS, preferred_element_type=_F32) + b2_ref[...]
    m = jnp.max(logits, axis=1, keepdims=True)
    lse = m + jnp.log(jnp.sum(jnp.exp(logits - m), axis=1, keepdims=True))
    logits_ref[...] = logits
    logp_ref[...] = logits - lse


def kernel(x, adj, W1, b1, W2, b2):
    n, nfeat = x.shape
    nhid = W1.shape[1]
    nclass = W2.shape[1]

    bm1 = 1024
    s1 = pl.pallas_call(
        _xw1_kernel,
        grid=(n // bm1,),
        in_specs=[
            pl.BlockSpec((bm1, nfeat), lambda i: (i, 0)),
            pl.BlockSpec((nfeat, nhid), lambda i: (0, 0)),
        ],
        out_specs=pl.BlockSpec((bm1, nhid), lambda i: (i, 0)),
        out_shape=jax.ShapeDtypeStruct((n, nhid), _BF),
    )(x, W1.astype(_BF))

    bm = 512
    h, s2 = pl.pallas_call(
        _layer1_kernel,
        grid=(n // bm,),
        in_specs=[
            pl.BlockSpec((bm, n), lambda i: (i, 0)),
            pl.BlockSpec((n, nhid), lambda i: (0, 0)),
            pl.BlockSpec((1, nhid), lambda i: (0, 0)),
            pl.BlockSpec((nhid, nclass), lambda i: (0, 0)),
        ],
        out_specs=[
            pl.BlockSpec((bm, nhid), lambda i: (i, 0)),
            pl.BlockSpec((bm, nclass), lambda i: (i, 0)),
        ],
        out_shape=[
            jax.ShapeDtypeStruct((n, nhid), _F32),
            jax.ShapeDtypeStruct((n, nclass), _BF),
        ],
    )(adj, s1, b1.reshape(1, nhid), W2.astype(_BF))

    logits, logp = pl.pallas_call(
        _layer2_kernel,
        grid=(n // bm,),
        in_specs=[
            pl.BlockSpec((bm, n), lambda i: (i, 0)),
            pl.BlockSpec((n, nclass), lambda i: (0, 0)),
            pl.BlockSpec((1, nclass), lambda i: (0, 0)),
        ],
        out_specs=[
            pl.BlockSpec((bm, nclass), lambda i: (i, 0)),
            pl.BlockSpec((bm, nclass), lambda i: (i, 0)),
        ],
        out_shape=[
            jax.ShapeDtypeStruct((n, nclass), _F32),
            jax.ShapeDtypeStruct((n, nclass), _F32),
        ],
    )(adj, s2, b2.reshape(1, nclass))

    return (logp, logits, h)


# speedup vs baseline: 1.1005x; 1.0245x over previous
import jax
import jax.numpy as jnp
from jax.experimental import pallas as pl
from jax.experimental.pallas import tpu as pltpu

_BF = jnp.bfloat16
_F32 = jnp.float32
_MM_DIMS = (((1,), (0,)), ((), ()))
_BM = 512


def _dot(a, b):
    return jax.lax.dot_general(a, b, _MM_DIMS, preferred_element_type=_F32)


def _mega_kernel(x_ref, adj_ref, w1_ref, b1_ref, w2_ref, b2_ref,
                 h_ref, logits_ref, logp_ref, s1_ref, s2_ref):
    i = pl.program_id(0)
    nblk = pl.num_programs(0) // 3

    @pl.when(i < nblk)
    def _phase_a():
        s1_ref[pl.ds((i % nblk) * _BM, _BM), :] = _dot(
            x_ref[...].astype(_BF), w1_ref[...]).astype(_BF)

    @pl.when((i >= nblk) & (i < 2 * nblk))
    def _phase_b():
        acc = _dot(adj_ref[...].astype(_BF), s1_ref[...])
        hblk = jnp.maximum(acc + b1_ref[...], 0.0)
        h_ref[...] = hblk
        s2_ref[pl.ds((i % nblk) * _BM, _BM), :] = _dot(
            hblk.astype(_BF), w2_ref[...]).astype(_BF)

    @pl.when(i >= 2 * nblk)
    def _phase_c():
        logits = _dot(adj_ref[...].astype(_BF), s2_ref[...]) + b2_ref[...]
        m = jnp.max(logits, axis=1, keepdims=True)
        lse = m + jnp.log(jnp.sum(jnp.exp(logits - m), axis=1, keepdims=True))
        logits_ref[...] = logits
        logp_ref[...] = logits - lse


def kernel(x, adj, W1, b1, W2, b2):
    n, nfeat = x.shape
    nhid = W1.shape[1]
    nclass = W2.shape[1]
    nblk = n // _BM

    def x_map(i):
        return (jnp.minimum(i, nblk - 1), 0)

    def adj_map(i):
        return (jnp.where(i < nblk, 0, i % nblk), 0)

    def h_map(i):
        return (jnp.clip(i - nblk, 0, nblk - 1), 0)

    def out_map(i):
        return (jnp.clip(i - 2 * nblk, 0, nblk - 1), 0)

    const = lambda i: (0, 0)

    h, logits, logp = pl.pallas_call(
        _mega_kernel,
        grid=(3 * nblk,),
        in_specs=[
            pl.BlockSpec((_BM, nfeat), x_map),
            pl.BlockSpec((_BM, n), adj_map),
            pl.BlockSpec((nfeat, nhid), const),
            pl.BlockSpec((1, nhid), const),
            pl.BlockSpec((nhid, nclass), const),
            pl.BlockSpec((1, nclass), const),
        ],
        out_specs=[
            pl.BlockSpec((_BM, nhid), h_map),
            pl.BlockSpec((_BM, nclass), out_map),
            pl.BlockSpec((_BM, nclass), out_map),
        ],
        out_shape=[
            jax.ShapeDtypeStruct((n, nhid), _F32),
            jax.ShapeDtypeStruct((n, nclass), _F32),
            jax.ShapeDtypeStruct((n, nclass), _F32),
        ],
        scratch_shapes=[
            pltpu.VMEM((n, nhid), _BF),
            pltpu.VMEM((n, nclass), _BF),
        ],
    )(x, adj, W1.astype(_BF), b1.reshape(1, nhid),
      W2.astype(_BF), b2.reshape(1, nclass))

    return (logp, logits, h)
